# baseline (speedup 1.0000x reference)
import functools
import math

import jax
import jax.numpy as jnp
import numpy as np
from jax.experimental import pallas as pl

_B = 8192
_SEQ = 512
_E = 8
_OUT = 96
_FFT = 512
_NB = _FFT // 2


_PAD = 128


def _moe_body(x_ref, F_ref, We_ref, Wg_ref, bg_ref, be_ref, cid_ref, o_ref):
    xb = x_ref[...]
    spec = jnp.dot(xb, F_ref[...], preferred_element_type=jnp.float32,
                   precision=jax.lax.Precision.HIGHEST)
    eo = jnp.dot(xb, We_ref[...], preferred_element_type=jnp.float32)
    eo = eo + be_ref[...]

    re = spec[:, :_NB]
    im = spec[:, _NB:]
    P = re * re + im * im
    kcol = jax.lax.broadcasted_iota(jnp.int32, P.shape, 1)
    P = jnp.where(kcol == 0, 0.0, P)
    s = jnp.sum(P, axis=1, keepdims=True)
    s = jnp.where(s == 0.0, 1.0, s)

    g = jnp.dot(P / s, Wg_ref[...], preferred_element_type=jnp.float32)
    g = g + bg_ref[...]

    ecol = jax.lax.broadcasted_iota(jnp.int32, g.shape, 1)
    m1 = jnp.max(g, axis=1, keepdims=True)
    a1 = jnp.min(jnp.where(g == m1, ecol, _E), axis=1, keepdims=True)
    gm = jnp.where(ecol == a1, -jnp.inf, g)
    m2 = jnp.max(gm, axis=1, keepdims=True)
    a2 = jnp.min(jnp.where(gm == m2, ecol, _E), axis=1, keepdims=True)
    e2 = jnp.exp(m2 - m1)
    denom = 1.0 + e2
    w1 = 1.0 / denom
    w2 = e2 / denom

    ccol = cid_ref[...]
    w_exp = jnp.where(ccol == a1, w1, 0.0) + jnp.where(ccol == a2, w2, 0.0)
    m = eo * w_exp
    acc = m[:, :_PAD]
    for e in range(1, _E):
        acc = acc + m[:, e * _PAD : (e + 1) * _PAD]
    o_ref[...] = acc[:, :_OUT]


@functools.partial(jax.jit, static_argnames=("bm", "interpret"))
def _run(x, Wg, bg, We, be, bm=512, interpret=False):
    n = _FFT
    s_idx = np.arange(_SEQ)[:, None]
    k_idx = np.arange(_NB)[None, :]
    ang = 2.0 * np.pi * s_idx * k_idx / n
    scale = 1.0 / math.sqrt(n)
    F = np.concatenate([np.cos(ang) * scale, -np.sin(ang) * scale], axis=1)
    F = jnp.asarray(F, dtype=jnp.float32)

    We_pad = jnp.zeros((_SEQ, _E, _PAD), jnp.float32)
    We_pad = We_pad.at[:, :, :_OUT].set(We.transpose(1, 0, 2))
    We_pad = We_pad.reshape(_SEQ, _E * _PAD)
    be_pad = jnp.zeros((_E, _PAD), jnp.float32).at[:, :_OUT].set(be)
    be_pad = be_pad.reshape(1, _E * _PAD)
    cid = jnp.asarray(np.repeat(np.arange(_E, dtype=np.int32), _PAD)[None, :])

    nrows = x.shape[0]
    grid = (nrows // bm,)
    out = pl.pallas_call(
        _moe_body,
        grid=grid,
        in_specs=[
            pl.BlockSpec((bm, _SEQ), lambda i: (i, 0)),
            pl.BlockSpec((_SEQ, 2 * _NB), lambda i: (0, 0)),
            pl.BlockSpec((_SEQ, _E * _PAD), lambda i: (0, 0)),
            pl.BlockSpec((_NB, _E), lambda i: (0, 0)),
            pl.BlockSpec((1, _E), lambda i: (0, 0)),
            pl.BlockSpec((1, _E * _PAD), lambda i: (0, 0)),
            pl.BlockSpec((1, _E * _PAD), lambda i: (0, 0)),
        ],
        out_specs=pl.BlockSpec((bm, _OUT), lambda i: (i, 0)),
        out_shape=jax.ShapeDtypeStruct((nrows, _OUT), jnp.float32),
        interpret=interpret,
    )(x, F, We_pad, Wg, bg.reshape(1, _E), be_pad, cid)
    return out


def kernel(x, Wg, bg, We, be):
    devs = jax.devices()
    nd = 2 if len(devs) >= 2 and _B % 2 == 0 else 1
    if nd == 1:
        return _run(x, Wg, bg, We, be)
    mesh = jax.sharding.Mesh(np.asarray(devs[:nd]), ("d",))
    P_ = jax.sharding.PartitionSpec
    f = jax.shard_map(
        lambda x_, Wg_, bg_, We_, be_: _run(x_, Wg_, bg_, We_, be_),
        mesh=mesh,
        in_specs=(P_("d", None), P_(None, None), P_(None),
                  P_(None, None, None), P_(None, None)),
        out_specs=P_("d", None),
        check_vma=False,
    )
    return f(x, Wg, bg, We, be)


# speedup vs baseline: 7.2767x; 7.2767x over previous
import functools
import math

import jax
import jax.numpy as jnp
import numpy as np
from jax.experimental import pallas as pl

_B = 8192
_SEQ = 512
_E = 8
_OUT = 96
_FFT = 512
_NB = _FFT // 2


_PAD = 128


def _moe_body(x_ref, F_ref, We_ref, Wg_ref, bg_ref, be_ref, cid_ref, o_ref):
    xb = x_ref[...]
    spec = jnp.dot(xb, F_ref[...], preferred_element_type=jnp.float32,
                   precision=jax.lax.Precision.HIGHEST)
    eo = jnp.dot(xb, We_ref[...], preferred_element_type=jnp.float32)
    eo = eo + be_ref[...]

    re = spec[:, :_NB]
    im = spec[:, _NB:]
    P = re * re + im * im
    kcol = jax.lax.broadcasted_iota(jnp.int32, P.shape, 1)
    P = jnp.where(kcol == 0, 0.0, P)
    s = jnp.sum(P, axis=1, keepdims=True)
    s = jnp.where(s == 0.0, 1.0, s)

    g = jnp.dot(P / s, Wg_ref[...], preferred_element_type=jnp.float32)
    g = g + bg_ref[...]

    ecol = jax.lax.broadcasted_iota(jnp.int32, g.shape, 1)
    m1 = jnp.max(g, axis=1, keepdims=True)
    a1 = jnp.min(jnp.where(g == m1, ecol, _E), axis=1, keepdims=True)
    gm = jnp.where(ecol == a1, -jnp.inf, g)
    m2 = jnp.max(gm, axis=1, keepdims=True)
    a2 = jnp.min(jnp.where(gm == m2, ecol, _E), axis=1, keepdims=True)
    e2 = jnp.exp(m2 - m1)
    denom = 1.0 + e2
    w1 = 1.0 / denom
    w2 = e2 / denom

    ccol = cid_ref[...]
    w_exp = jnp.where(ccol == a1, w1, 0.0) + jnp.where(ccol == a2, w2, 0.0)
    m = eo * w_exp
    acc = m[:, :_PAD]
    for e in range(1, _E):
        acc = acc + m[:, e * _PAD : (e + 1) * _PAD]
    o_ref[...] = acc[:, :_OUT]


@functools.partial(jax.jit, static_argnames=("bm", "interpret"))
def _run(x, Wg, bg, We, be, bm=512, interpret=False):
    n = _FFT
    s_idx = np.arange(_SEQ)[:, None]
    k_idx = np.arange(_NB)[None, :]
    ang = 2.0 * np.pi * s_idx * k_idx / n
    scale = 1.0 / math.sqrt(n)
    F = np.concatenate([np.cos(ang) * scale, -np.sin(ang) * scale], axis=1)
    F = jnp.asarray(F, dtype=jnp.float32)

    We_pad = jnp.zeros((_SEQ, _E, _PAD), jnp.float32)
    We_pad = We_pad.at[:, :, :_OUT].set(We.transpose(1, 0, 2))
    We_pad = We_pad.reshape(_SEQ, _E * _PAD)
    be_pad = jnp.zeros((_E, _PAD), jnp.float32).at[:, :_OUT].set(be)
    be_pad = be_pad.reshape(1, _E * _PAD)
    cid = jnp.asarray(np.repeat(np.arange(_E, dtype=np.int32), _PAD)[None, :])

    nrows = x.shape[0]
    grid = (nrows // bm,)
    out = pl.pallas_call(
        _moe_body,
        grid=grid,
        in_specs=[
            pl.BlockSpec((bm, _SEQ), lambda i: (i, 0)),
            pl.BlockSpec((_SEQ, 2 * _NB), lambda i: (0, 0)),
            pl.BlockSpec((_SEQ, _E * _PAD), lambda i: (0, 0)),
            pl.BlockSpec((_NB, _E), lambda i: (0, 0)),
            pl.BlockSpec((1, _E), lambda i: (0, 0)),
            pl.BlockSpec((1, _E * _PAD), lambda i: (0, 0)),
            pl.BlockSpec((1, _E * _PAD), lambda i: (0, 0)),
        ],
        out_specs=pl.BlockSpec((bm, _OUT), lambda i: (i, 0)),
        out_shape=jax.ShapeDtypeStruct((nrows, _OUT), jnp.float32),
        interpret=interpret,
    )(x, F, We_pad, Wg, bg.reshape(1, _E), be_pad, cid)
    return out


def kernel(x, Wg, bg, We, be):
    return _run(x, Wg, bg, We, be)


# speedup vs baseline: 7.7151x; 1.0603x over previous
import functools
import math

import jax
import jax.numpy as jnp
import numpy as np
from jax.experimental import pallas as pl

_B = 8192
_SEQ = 512
_E = 8
_OUT = 96
_FFT = 512
_NB = _FFT // 2


_PAD = 128


def _moe_body(x_ref, F_ref, We_ref, Wg_ref, bg_ref, be_ref, cid_ref, o_ref):
    xb = x_ref[...]
    spec = jnp.dot(xb, F_ref[...], preferred_element_type=jnp.float32,
                   precision=jax.lax.Precision.HIGHEST)
    eo = jnp.dot(xb, We_ref[...], preferred_element_type=jnp.float32)
    eo = eo + be_ref[...]

    re = spec[:, :_NB]
    im = spec[:, _NB:]
    P = re * re + im * im
    kcol = jax.lax.broadcasted_iota(jnp.int32, P.shape, 1)
    P = jnp.where(kcol == 0, 0.0, P)
    s = jnp.sum(P, axis=1, keepdims=True)
    s = jnp.where(s == 0.0, 1.0, s)

    g = jnp.dot(P / s, Wg_ref[...], preferred_element_type=jnp.float32)
    g = g + bg_ref[...]

    ecol = jax.lax.broadcasted_iota(jnp.int32, g.shape, 1)
    m1 = jnp.max(g, axis=1, keepdims=True)
    a1 = jnp.min(jnp.where(g == m1, ecol, _E), axis=1, keepdims=True)
    gm = jnp.where(ecol == a1, -jnp.inf, g)
    m2 = jnp.max(gm, axis=1, keepdims=True)
    a2 = jnp.min(jnp.where(gm == m2, ecol, _E), axis=1, keepdims=True)
    e2 = jnp.exp(m2 - m1)
    denom = 1.0 + e2
    w1 = 1.0 / denom
    w2 = e2 / denom

    ccol = cid_ref[...]
    w_exp = jnp.where(ccol == a1, w1, 0.0) + jnp.where(ccol == a2, w2, 0.0)
    m = eo * w_exp
    acc = m[:, :_PAD]
    for e in range(1, _E):
        acc = acc + m[:, e * _PAD : (e + 1) * _PAD]
    o_ref[...] = acc[:, :_OUT]


@functools.partial(jax.jit, static_argnames=("bm", "interpret"))
def _run(x, Wg, bg, We, be, bm=2048, interpret=False):
    n = _FFT
    s_idx = np.arange(_SEQ)[:, None]
    k_idx = np.arange(_NB)[None, :]
    ang = 2.0 * np.pi * s_idx * k_idx / n
    scale = 1.0 / math.sqrt(n)
    F = np.concatenate([np.cos(ang) * scale, -np.sin(ang) * scale], axis=1)
    F = jnp.asarray(F, dtype=jnp.float32)

    We_pad = jnp.zeros((_SEQ, _E, _PAD), jnp.float32)
    We_pad = We_pad.at[:, :, :_OUT].set(We.transpose(1, 0, 2))
    We_pad = We_pad.reshape(_SEQ, _E * _PAD)
    be_pad = jnp.zeros((_E, _PAD), jnp.float32).at[:, :_OUT].set(be)
    be_pad = be_pad.reshape(1, _E * _PAD)
    cid = jnp.asarray(np.repeat(np.arange(_E, dtype=np.int32), _PAD)[None, :])

    nrows = x.shape[0]
    grid = (nrows // bm,)
    out = pl.pallas_call(
        _moe_body,
        grid=grid,
        in_specs=[
            pl.BlockSpec((bm, _SEQ), lambda i: (i, 0)),
            pl.BlockSpec((_SEQ, 2 * _NB), lambda i: (0, 0)),
            pl.BlockSpec((_SEQ, _E * _PAD), lambda i: (0, 0)),
            pl.BlockSpec((_NB, _E), lambda i: (0, 0)),
            pl.BlockSpec((1, _E), lambda i: (0, 0)),
            pl.BlockSpec((1, _E * _PAD), lambda i: (0, 0)),
            pl.BlockSpec((1, _E * _PAD), lambda i: (0, 0)),
        ],
        out_specs=pl.BlockSpec((bm, _OUT), lambda i: (i, 0)),
        out_shape=jax.ShapeDtypeStruct((nrows, _OUT), jnp.float32),
        interpret=interpret,
    )(x, F, We_pad, Wg, bg.reshape(1, _E), be_pad, cid)
    return out


def kernel(x, Wg, bg, We, be):
    return _run(x, Wg, bg, We, be)
